# baseline (speedup 1.0000x reference)
import functools

import jax
import jax.numpy as jnp
from jax import lax
from jax.experimental import pallas as pl
from jax.experimental.pallas import tpu as pltpu
from jax.experimental.pallas import tpu_sc as plsc

VOCAB = 1000000
DIM = 16
NCLS = 6
B = 16384
NNEG = 20

NC = 2
NS = 16
NW = NC * NS
CH = 128
NBUF = 4

U_CHUNKS = B // NW // CH
N_CHUNKS = B * NNEG // NW // CH
U_ROWS = B // NW
N_ROWS = B * NNEG // NW

@functools.cache
def _make_sc_gather():
    mesh = plsc.VectorSubcoreMesh(
        core_axis_name="c", subcore_axis_name="s", num_cores=NC, num_subcores=NS
    )
    return functools.partial(
        pl.kernel,
        out_type=(
            jax.ShapeDtypeStruct((B, DIM), jnp.float32),
            jax.ShapeDtypeStruct((B, DIM), jnp.float32),
            jax.ShapeDtypeStruct((B * NNEG, DIM), jnp.float32),
        ),
        mesh=mesh,
        compiler_params=pltpu.CompilerParams(use_tc_tiling_on_sc=False),
        scratch_types=[
            pltpu.VMEM((U_CHUNKS, CH), jnp.int32),
            pltpu.VMEM((U_CHUNKS, CH), jnp.int32),
            pltpu.VMEM((N_CHUNKS, CH), jnp.int32),
            pltpu.VMEM((NBUF, CH, DIM), jnp.float32),
            pltpu.SemaphoreType.DMA,
            pltpu.SemaphoreType.DMA,
            pltpu.SemaphoreType.DMA,
            pltpu.SemaphoreType.DMA,
        ],
    )(_sc_gather_body)


def _sc_gather_body(u_hbm, v_hbm, pu_hbm, pv_hbm, ng_hbm, eu_hbm, ev_hbm, en_hbm,
                    idx_u, idx_v, idx_n, bufs, s0, s1, s2, s3):
    sems = (s0, s1, s2, s3)
    wid = lax.axis_index("s") * NC + lax.axis_index("c")

    pltpu.sync_copy(pu_hbm.at[wid], idx_u)
    pltpu.sync_copy(pv_hbm.at[wid], idx_v)
    pltpu.sync_copy(ng_hbm.at[wid], idx_n)

    def seg(tbl, idx2, out, nchunks, base_rows):
        for s in range(NBUF):
            pltpu.async_copy(tbl.at[idx2.at[s]], bufs.at[s], sems[s])

        @pl.loop(0, nchunks // NBUF)
        def _group(g):
            for s in range(NBUF):
                j = g * NBUF + s
                pltpu.make_async_copy(tbl.at[idx2.at[j]], bufs.at[s], sems[s]).wait()
                pltpu.sync_copy(bufs.at[s], out.at[pl.ds(base_rows + j * CH, CH)])

                @pl.when(j + NBUF < nchunks)
                def _():
                    pltpu.async_copy(tbl.at[idx2.at[j + NBUF]], bufs.at[s], sems[s])

    seg(u_hbm, idx_u, eu_hbm, U_CHUNKS, wid * U_ROWS)
    seg(v_hbm, idx_v, ev_hbm, U_CHUNKS, wid * U_ROWS)
    seg(v_hbm, idx_n, en_hbm, N_CHUNKS, wid * N_ROWS)


RB = 256


def _tc_body(u_ref, v_ref, n_ref, wt_ref, b_ref, loss_ref, dur_ref):
    i = pl.program_id(0)
    u = u_ref[...]
    v = v_ref[...]
    n3 = n_ref[...]

    score = jnp.sum(u * v, axis=1, keepdims=True)
    score = jnp.clip(score, -10.0, 10.0)
    pos_l = jnp.log1p(jnp.exp(-score))

    nd = jnp.sum(n3 * u[:, None, :], axis=2)
    nd = jnp.clip(nd, -10.0, 10.0)
    neg_l = jnp.sum(jnp.log1p(jnp.exp(nd)), axis=1, keepdims=True)

    part = jnp.sum(pos_l + neg_l) * (1.0 / B)

    @pl.when(i == 0)
    def _():
        loss_ref[...] = jnp.zeros((1, 1), jnp.float32)

    loss_ref[...] = loss_ref[...] + jnp.full((1, 1), part, jnp.float32)
    dur_ref[...] = (
        jnp.dot(u, wt_ref[...], preferred_element_type=jnp.float32) + b_ref[...]
    )


_tc_compute = pl.pallas_call(
    _tc_body,
    grid=(B // RB,),
    in_specs=[
        pl.BlockSpec((RB, DIM), lambda i: (i, 0)),
        pl.BlockSpec((RB, DIM), lambda i: (i, 0)),
        pl.BlockSpec((RB, NNEG, DIM), lambda i: (i, 0, 0)),
        pl.BlockSpec((DIM, NCLS), lambda i: (0, 0)),
        pl.BlockSpec((1, NCLS), lambda i: (0, 0)),
    ],
    out_specs=[
        pl.BlockSpec((1, 1), lambda i: (0, 0)),
        pl.BlockSpec((RB, NCLS), lambda i: (i, 0)),
    ],
    out_shape=[
        jax.ShapeDtypeStruct((1, 1), jnp.float32),
        jax.ShapeDtypeStruct((B, NCLS), jnp.float32),
    ],
)


def kernel(u_emb, v_emb, W, b, pos_u, pos_v, neg_v):
    pu = pos_u.astype(jnp.int32).reshape(NW, U_CHUNKS, CH)
    pv = pos_v.astype(jnp.int32).reshape(NW, U_CHUNKS, CH)
    ng = neg_v.astype(jnp.int32).reshape(NW, N_CHUNKS, CH)
    emb_u, emb_v, emb_neg = _make_sc_gather()(u_emb, v_emb, pu, pv, ng)
    loss_arr, dur = _tc_compute(
        emb_u, emb_v, emb_neg.reshape(B, NNEG, DIM),
        W.T, b.reshape(1, NCLS),
    )
    return loss_arr[0, 0], dur
